# baseline (speedup 1.0000x reference)
import jax
import jax.numpy as jnp
from jax import lax
from jax.experimental import pallas as pl
from jax.experimental.pallas import tpu as pltpu
from jax.experimental.pallas import tpu_sc as plsc

TOTAL = 32768
B = 16
IN_CH = 4
HID = 64
FEAT = 16
OUT = 256
P = 3.0
EPS = 1e-6

CHUNK = 16384
NUM = TOTAL // CHUNK
L = 16


def _mlp_kernel(featsT_ref, w1t_ref, w2t_ref, xp_ref):
    x = jnp.maximum(
        jnp.dot(w1t_ref[...], featsT_ref[...],
                preferred_element_type=jnp.float32), 0.0)
    x = jnp.dot(w2t_ref[...], x,
                preferred_element_type=jnp.float32)
    xc = jnp.maximum(x, EPS)
    xp_ref[...] = xc * xc * xc


def _cbrt(x):
    i = plsc.bitcast(x, jnp.int32)
    y = plsc.bitcast(i // 3 + 709952852, jnp.float32)
    for _ in range(5):
        y = (2.0 * y + x / (y * y)) * (1.0 / 3.0)
    return y


def _sc_gem_head(xpT_hbm, ids_hbm, wfc_hbm, out_hbm,
                 vals_v, ids_v, starts_v, accmat_v, tmp_v,
                 gcol_v, gemt_v, wfc_v, row_v, sh_starts, sh_gem):
    c = lax.axis_index("c")
    s = lax.axis_index("s")

    @pl.when(c == 0)
    def _core0():
        pltpu.sync_copy(xpT_hbm.at[s], vals_v)

        @pl.when(s == 0)
        def _bounds():
            pltpu.sync_copy(ids_hbm, ids_v)
            lanes = lax.iota(jnp.int32, L)

            def first_geq(tgt):
                def body(_, lh):
                    lo, hi = lh
                    mid = (lo + hi) // 2
                    v = plsc.load_gather(ids_v, [mid])
                    pred = v >= tgt
                    return (jnp.where(pred, lo, mid + 1),
                            jnp.where(pred, mid, hi))
                lo, _ = lax.fori_loop(
                    0, 15, body,
                    (jnp.zeros((L,), jnp.int32),
                     jnp.full((L,), TOTAL, jnp.int32)))
                return lo

            starts_v[...] = first_geq(lanes)
            pltpu.sync_copy(starts_v, sh_starts)

        plsc.subcore_barrier()
        pltpu.sync_copy(sh_starts, starts_v)

        lanes = lax.iota(jnp.int32, L)
        sv = starts_v[...]
        nxt = jnp.minimum(lanes + 1, L - 1)
        ev = jnp.where(lanes == L - 1, TOTAL,
                       plsc.load_gather(starts_v, [nxt]))

        for b in range(B):
            lo = sv[b]
            hi = ev[b]
            j0 = (lo // L) * L
            nst = (hi - j0 + (L - 1)) // L
            lob = lax.broadcast(lo, (L,))
            hib = lax.broadcast(hi, (L,))

            def body(k, acc, j0=j0, lob=lob, hib=hib):
                j = j0 + k * L
                v = vals_v[pl.ds(j, L)]
                lane = lax.iota(jnp.int32, L) + lax.broadcast(j, (L,))
                m = (lane >= lob) & (lane < hib)
                return acc + jnp.where(m, v, 0.0)

            acc = lax.fori_loop(0, nst, body, jnp.zeros((L,), jnp.float32))
            accmat_v[pl.ds(b * L, L)] = acc

        lanesL = lanes * L
        seg = jnp.zeros((L,), jnp.float32)
        for k in range(L):
            seg = seg + plsc.load_gather(accmat_v, [lanesL + k])

        cntf = (ev - sv).astype(jnp.float32)
        mx = cntf
        for sh in (1, 2, 4, 8):
            tmp_v[...] = mx
            mx = jnp.maximum(
                mx, plsc.load_gather(tmp_v, [(lanes + sh) & (L - 1)]))
        xg = (seg + (mx - cntf) * (EPS ** 3)) / mx
        gcol_v[...] = _cbrt(xg)
        pltpu.sync_copy(gcol_v, sh_gem.at[pl.ds(s * L, L)])
        plsc.subcore_barrier()

        pltpu.sync_copy(sh_gem, gemt_v)
        pltpu.sync_copy(wfc_hbm, wfc_v)
        gidx = lax.iota(jnp.int32, L) * L + lax.broadcast(s, (L,))
        gv = plsc.load_gather(gemt_v, [gidx])
        accs = [jnp.zeros((L,), jnp.float32) for _ in range(OUT // L)]
        for f in range(FEAT):
            gfb = lax.broadcast(gv[f], (L,))
            for j in range(OUT // L):
                accs[j] += gfb * wfc_v[f, pl.ds(j * L, L)]
        for j in range(OUT // L):
            row_v[pl.ds(j * L, L)] = accs[j]
        pltpu.sync_copy(row_v, out_hbm.at[s])


@jax.jit
def kernel(feats, batch_ids, W1, W2, Wfc):
    featsT = feats.T
    xpT = pl.pallas_call(
        _mlp_kernel,
        grid=(NUM,),
        in_specs=[
            pl.BlockSpec((IN_CH, CHUNK), lambda i: (0, i)),
            pl.BlockSpec((HID, IN_CH), lambda i: (0, 0)),
            pl.BlockSpec((FEAT, HID), lambda i: (0, 0)),
        ],
        out_specs=pl.BlockSpec((FEAT, CHUNK), lambda i: (0, i)),
        out_shape=jax.ShapeDtypeStruct((FEAT, TOTAL), jnp.float32),
    )(featsT, W1.T, W2.T)

    mesh = plsc.VectorSubcoreMesh(core_axis_name="c", subcore_axis_name="s")
    sc_fn = pl.kernel(
        _sc_gem_head,
        mesh=mesh,
        out_type=jax.ShapeDtypeStruct((B, OUT), jnp.float32),
        compiler_params=pltpu.CompilerParams(needs_layout_passes=False),
        scratch_types=[
            pltpu.VMEM((TOTAL,), jnp.float32),
            pltpu.VMEM((TOTAL,), jnp.int32),
            pltpu.VMEM((L,), jnp.int32),
            pltpu.VMEM((B * L,), jnp.float32),
            pltpu.VMEM((L,), jnp.float32),
            pltpu.VMEM((L,), jnp.float32),
            pltpu.VMEM((B * FEAT,), jnp.float32),
            pltpu.VMEM((FEAT, OUT), jnp.float32),
            pltpu.VMEM((OUT,), jnp.float32),
            pltpu.VMEM_SHARED((L,), jnp.int32),
            pltpu.VMEM_SHARED((B * FEAT,), jnp.float32),
        ],
    )
    return sc_fn(xpT, batch_ids, Wfc)


# speedup vs baseline: 1.1512x; 1.1512x over previous
import jax
import jax.numpy as jnp
from jax import lax
from jax.experimental import pallas as pl
from jax.experimental.pallas import tpu as pltpu
from jax.experimental.pallas import tpu_sc as plsc

TOTAL = 32768
B = 16
IN_CH = 4
HID = 64
FEAT = 16
OUT = 256
P = 3.0
EPS = 1e-6

CHUNK = 16384
NUM = TOTAL // CHUNK
L = 16


def _mlp_kernel(featsT_ref, w1t_ref, w2t_ref, xp_ref):
    x = jnp.maximum(
        jnp.dot(w1t_ref[...], featsT_ref[...],
                preferred_element_type=jnp.float32), 0.0)
    x = jnp.dot(w2t_ref[...], x,
                preferred_element_type=jnp.float32)
    xc = jnp.maximum(x, EPS)
    xp_ref[...] = xc * xc * xc


def _cbrt(x):
    i = plsc.bitcast(x, jnp.int32)
    y = plsc.bitcast(i // 3 + 709952852, jnp.float32)
    for _ in range(5):
        y = (2.0 * y + x / (y * y)) * (1.0 / 3.0)
    return y


def _sc_gem_head(xpT_hbm, ids_hbm, wfc_hbm, out_hbm,
                 vals_v, ids_v, starts_v, accmat_v, tmp_v,
                 gcol_v, gemt_v, wfc_v, row_v, sh_starts, sh_gem, wfc_sem):
    c = lax.axis_index("c")
    s = lax.axis_index("s")

    @pl.when(c == 0)
    def _core0():
        wfc_cp = pltpu.async_copy(wfc_hbm, wfc_v, wfc_sem)
        pltpu.sync_copy(xpT_hbm.at[s], vals_v)

        @pl.when(s == 0)
        def _bounds():
            pltpu.sync_copy(ids_hbm, ids_v)
            lanes = lax.iota(jnp.int32, L)

            def first_geq(tgt):
                def body(_, lh):
                    lo, hi = lh
                    mid = (lo + hi) // 2
                    v = plsc.load_gather(ids_v, [mid])
                    pred = v >= tgt
                    return (jnp.where(pred, lo, mid + 1),
                            jnp.where(pred, mid, hi))
                lo, _ = lax.fori_loop(
                    0, 15, body,
                    (jnp.zeros((L,), jnp.int32),
                     jnp.full((L,), TOTAL, jnp.int32)))
                return lo

            starts_v[...] = first_geq(lanes)
            pltpu.sync_copy(starts_v, sh_starts)

        plsc.subcore_barrier()
        pltpu.sync_copy(sh_starts, starts_v)

        lanes = lax.iota(jnp.int32, L)
        sv = starts_v[...]
        nxt = jnp.minimum(lanes + 1, L - 1)
        ev = jnp.where(lanes == L - 1, TOTAL,
                       plsc.load_gather(starts_v, [nxt]))

        for b in range(B):
            lo = sv[b]
            hi = ev[b]
            ja = (lo + L - 1) // L
            jb = hi // L

            lj = jnp.minimum((lo // L) * L, TOTAL - L)
            lane = lax.iota(jnp.int32, L) + lax.broadcast(lj, (L,))
            lm = ((lane >= lax.broadcast(lo, (L,))) &
                  (lane < lax.broadcast(jnp.minimum(ja * L, hi), (L,))))
            acc = jnp.where(lm, vals_v[pl.ds(lj, L)], 0.0)

            t0 = jnp.maximum(ja, jb) * L
            tj = jnp.minimum(t0, TOTAL - L)
            lane = lax.iota(jnp.int32, L) + lax.broadcast(tj, (L,))
            tm = ((lane >= lax.broadcast(t0, (L,))) &
                  (lane < lax.broadcast(hi, (L,))))
            acc = acc + jnp.where(tm, vals_v[pl.ds(tj, L)], 0.0)

            nin = jnp.maximum(jb - ja, 0)
            n4 = nin // 4

            def body4(k, accs, ja=ja):
                a0, a1, a2, a3 = accs
                j = (ja + 4 * k) * L
                return (a0 + vals_v[pl.ds(j, L)],
                        a1 + vals_v[pl.ds(j + L, L)],
                        a2 + vals_v[pl.ds(j + 2 * L, L)],
                        a3 + vals_v[pl.ds(j + 3 * L, L)])

            z = jnp.zeros((L,), jnp.float32)
            a0, a1, a2, a3 = lax.fori_loop(0, n4, body4, (z, z, z, z))

            def body1(k, acc1, ja=ja, n4=n4):
                j = (ja + 4 * n4 + k) * L
                return acc1 + vals_v[pl.ds(j, L)]

            acc = acc + lax.fori_loop(0, nin - 4 * n4, body1,
                                      (a0 + a1) + (a2 + a3))
            accmat_v[pl.ds(b * L, L)] = acc

        lanesL = lanes * L
        seg = jnp.zeros((L,), jnp.float32)
        for k in range(L):
            seg = seg + plsc.load_gather(accmat_v, [lanesL + k])

        cntf = (ev - sv).astype(jnp.float32)
        mx = cntf
        for sh in (1, 2, 4, 8):
            tmp_v[...] = mx
            mx = jnp.maximum(
                mx, plsc.load_gather(tmp_v, [(lanes + sh) & (L - 1)]))
        xg = (seg + (mx - cntf) * (EPS ** 3)) / mx
        gcol_v[...] = _cbrt(xg)
        pltpu.sync_copy(gcol_v, sh_gem.at[pl.ds(s * L, L)])
        plsc.subcore_barrier()

        pltpu.sync_copy(sh_gem, gemt_v)
        wfc_cp.wait()
        gidx = lax.iota(jnp.int32, L) * L + lax.broadcast(s, (L,))
        gv = plsc.load_gather(gemt_v, [gidx])
        accs = [jnp.zeros((L,), jnp.float32) for _ in range(OUT // L)]
        for f in range(FEAT):
            gfb = lax.broadcast(gv[f], (L,))
            for j in range(OUT // L):
                accs[j] += gfb * wfc_v[f, pl.ds(j * L, L)]
        for j in range(OUT // L):
            row_v[pl.ds(j * L, L)] = accs[j]
        pltpu.sync_copy(row_v, out_hbm.at[s])


@jax.jit
def kernel(feats, batch_ids, W1, W2, Wfc):
    featsT = feats.T
    xpT = pl.pallas_call(
        _mlp_kernel,
        grid=(NUM,),
        in_specs=[
            pl.BlockSpec((IN_CH, CHUNK), lambda i: (0, i)),
            pl.BlockSpec((HID, IN_CH), lambda i: (0, 0)),
            pl.BlockSpec((FEAT, HID), lambda i: (0, 0)),
        ],
        out_specs=pl.BlockSpec((FEAT, CHUNK), lambda i: (0, i)),
        out_shape=jax.ShapeDtypeStruct((FEAT, TOTAL), jnp.float32),
    )(featsT, W1.T, W2.T)

    mesh = plsc.VectorSubcoreMesh(core_axis_name="c", subcore_axis_name="s")
    sc_fn = pl.kernel(
        _sc_gem_head,
        mesh=mesh,
        out_type=jax.ShapeDtypeStruct((B, OUT), jnp.float32),
        compiler_params=pltpu.CompilerParams(needs_layout_passes=False),
        scratch_types=[
            pltpu.VMEM((TOTAL,), jnp.float32),
            pltpu.VMEM((TOTAL,), jnp.int32),
            pltpu.VMEM((L,), jnp.int32),
            pltpu.VMEM((B * L,), jnp.float32),
            pltpu.VMEM((L,), jnp.float32),
            pltpu.VMEM((L,), jnp.float32),
            pltpu.VMEM((B * FEAT,), jnp.float32),
            pltpu.VMEM((FEAT, OUT), jnp.float32),
            pltpu.VMEM((OUT,), jnp.float32),
            pltpu.VMEM_SHARED((L,), jnp.int32),
            pltpu.VMEM_SHARED((B * FEAT,), jnp.float32),
            pltpu.SemaphoreType.DMA,
        ],
    )
    return sc_fn(xpT, batch_ids, Wfc)


# speedup vs baseline: 5.7239x; 4.9721x over previous
import jax
import jax.numpy as jnp
from jax.experimental import pallas as pl

TOTAL = 32768
B = 16
IN_CH = 4
HID = 64
FEAT = 16
OUT = 256
P = 3.0
EPS = 1e-6


def _gem_kernel(featsT_ref, ids_ref, w1t_ref, w2t_ref, wfc_ref, out_ref):
    x = jnp.maximum(
        jnp.dot(w1t_ref[...], featsT_ref[...],
                preferred_element_type=jnp.float32), 0.0)
    x = jnp.dot(w2t_ref[...], x,
                preferred_element_type=jnp.float32)
    xc = jnp.maximum(x, EPS)
    xp = xc * xc * xc
    xp_ext = jnp.pad(xp, ((0, 1), (0, 0)), constant_values=1.0)

    ids = ids_ref[0, 0, :]
    onehot = (ids[None, :] == jax.lax.broadcasted_iota(
        jnp.int32, (B, TOTAL), 0)).astype(jnp.float32)
    seg = jax.lax.dot_general(
        onehot, xp_ext, dimension_numbers=(((1,), (1,)), ((), ())),
        preferred_element_type=jnp.float32)

    cnt = seg[:, FEAT]
    max_len = jnp.max(cnt)
    pad = (max_len - cnt)[:, None] * (EPS ** 3)
    gem = jnp.power((seg[:, :FEAT] + pad) / max_len, 1.0 / 3.0)
    out_ref[...] = jnp.dot(gem, wfc_ref[...],
                           preferred_element_type=jnp.float32)


@jax.jit
def kernel(feats, batch_ids, W1, W2, Wfc):
    ids3 = batch_ids.reshape(1, 1, TOTAL)
    return pl.pallas_call(
        _gem_kernel,
        out_shape=jax.ShapeDtypeStruct((B, OUT), jnp.float32),
    )(feats.T, ids3, W1.T, W2.T, Wfc)


# speedup vs baseline: 6.3015x; 1.1009x over previous
import jax
import jax.numpy as jnp
from jax.experimental import pallas as pl
from jax.experimental.pallas import tpu as pltpu

TOTAL = 32768
B = 16
IN_CH = 4
HID = 64
FEAT = 16
OUT = 256
P = 3.0
EPS = 1e-6

CHUNK = 32768
NUM = TOTAL // CHUNK


def _gem_kernel(featsT_ref, ids_ref, w1t_ref, w2t_ref, wfc_ref, out_ref,
                seg_ref):
    i = pl.program_id(0)

    @pl.when(i == 0)
    def _init():
        seg_ref[...] = jnp.zeros_like(seg_ref)

    x = jnp.maximum(
        jnp.dot(w1t_ref[...], featsT_ref[...],
                preferred_element_type=jnp.float32), 0.0)
    x = jnp.dot(w2t_ref[...], x,
                preferred_element_type=jnp.float32)
    xc = jnp.maximum(x, EPS)
    xp = xc * xc * xc
    xp_ext = jnp.pad(xp, ((0, 1), (0, 0)), constant_values=1.0)

    ids = ids_ref[0, 0, :]
    onehot = (ids[None, :] == jax.lax.broadcasted_iota(
        jnp.int32, (B, CHUNK), 0)).astype(jnp.float32)
    seg_ref[...] += jax.lax.dot_general(
        onehot, xp_ext, dimension_numbers=(((1,), (1,)), ((), ())),
        preferred_element_type=jnp.float32)

    @pl.when(i == NUM - 1)
    def _finish():
        cnt = seg_ref[:, FEAT]
        max_len = jnp.max(cnt)
        pad = (max_len - cnt)[:, None] * (EPS ** 3)
        gem = jnp.power((seg_ref[:, :FEAT] + pad) / max_len, 1.0 / 3.0)
        out_ref[...] = jnp.dot(gem, wfc_ref[...],
                               preferred_element_type=jnp.float32)


@jax.jit
def kernel(feats, batch_ids, W1, W2, Wfc):
    ids3 = batch_ids.reshape(NUM, 1, CHUNK)
    featsT = feats.T
    return pl.pallas_call(
        _gem_kernel,
        grid=(NUM,),
        in_specs=[
            pl.BlockSpec((IN_CH, CHUNK), lambda i: (0, i)),
            pl.BlockSpec((1, 1, CHUNK), lambda i: (i, 0, 0)),
            pl.BlockSpec((HID, IN_CH), lambda i: (0, 0)),
            pl.BlockSpec((FEAT, HID), lambda i: (0, 0)),
            pl.BlockSpec((FEAT, OUT), lambda i: (0, 0)),
        ],
        out_specs=pl.BlockSpec((B, OUT), lambda i: (0, 0)),
        out_shape=jax.ShapeDtypeStruct((B, OUT), jnp.float32),
        scratch_shapes=[
            pltpu.VMEM((B, FEAT + 1), jnp.float32),
        ],
    )(featsT, ids3, W1.T, W2.T, Wfc)
